# baseline (speedup 1.0000x reference)
import functools

import jax
import jax.numpy as jnp
from jax import lax
from jax.experimental import pallas as pl
from jax.experimental.pallas import tpu as pltpu
from jax.experimental.pallas import tpu_sc as plsc

_START = 101
_END = 102
_SEP = 102
_HALF = 2048
_LANES = 16
_CHUNKS = _HALF // _LANES

_MESH = plsc.VectorSubcoreMesh(core_axis_name="c", subcore_axis_name="s")


@functools.partial(
    pl.kernel,
    mesh=_MESH,
    out_type=[
        jax.ShapeDtypeStruct((16, 2 * _HALF), jnp.int32),
        jax.ShapeDtypeStruct((16, 2 * _HALF), jnp.int32),
    ],
    scratch_types=[
        pltpu.VMEM((_HALF,), jnp.int32),
        pltpu.VMEM((_HALF,), jnp.int32),
        pltpu.VMEM((_HALF,), jnp.int32),
        pltpu.SemaphoreType.DMA,
        pltpu.SemaphoreType.DMA,
    ],
    compiler_params=pltpu.CompilerParams(
        needs_layout_passes=False, skip_device_barrier=True
    ),
)
def _pack_sc(seg1, seg2, tok_out, sid_out, src_v, tok_v, sid_v, sem_in, sem_sid):
    half = lax.axis_index("c")
    row = lax.axis_index("s")
    col0 = half * _HALF

    @pl.when(half == 0)
    def _():
        pltpu.async_copy(seg1.at[row], src_v, sem_in)

    @pl.when(half == 1)
    def _():
        pltpu.async_copy(seg2.at[row], src_v, sem_in)

    lane = lax.iota(jnp.int32, _LANES)
    first_val = jnp.where(half == 0, jnp.int32(_START), jnp.int32(_SEP))
    is_second = (half == 1).astype(jnp.int32)

    sid_v[pl.ds(0, _LANES)] = jnp.where(lane == 0, jnp.int32(0), is_second)
    sid_fill = jnp.broadcast_to(is_second, (_LANES,))

    @plsc.parallel_loop(1, _CHUNKS, unroll=1)
    def _(j):
        sid_v[pl.ds(j * _LANES, _LANES)] = sid_fill

    sid_cp = pltpu.async_copy(sid_v, sid_out.at[row, pl.ds(col0, _HALF)], sem_sid)

    pltpu.make_async_copy(seg1.at[row], src_v, sem_in).wait()

    v0 = plsc.load_gather(src_v, [jnp.maximum(lane - 1, 0)])
    tok_v[pl.ds(0, _LANES)] = jnp.where(lane == 0, first_val, v0)

    @plsc.parallel_loop(1, _CHUNKS, unroll=1)
    def _(j):
        p = lane + j * _LANES
        v = plsc.load_gather(src_v, [p - 1])
        tok_v[pl.ds(j * _LANES, _LANES)] = v

    tail0 = _HALF - _LANES
    vt = tok_v[pl.ds(tail0, _LANES)]
    fix_end = (lane == _LANES - 1) & (half == 1)
    tok_v[pl.ds(tail0, _LANES)] = jnp.where(fix_end, jnp.int32(_END), vt)

    pltpu.sync_copy(tok_v, tok_out.at[row, pl.ds(col0, _HALF)])
    sid_cp.wait()


def kernel(seg1, seg2):
    tokens, segment_ids = _pack_sc(seg1, seg2)
    return tokens, segment_ids


# speedup vs baseline: 1.0059x; 1.0059x over previous
import functools

import jax
import jax.numpy as jnp
from jax import lax
from jax.experimental import pallas as pl
from jax.experimental.pallas import tpu as pltpu
from jax.experimental.pallas import tpu_sc as plsc

_START = 101
_END = 102
_SEP = 102
_HALF = 2048
_LANES = 16
_CHUNKS = _HALF // _LANES

_MESH = plsc.VectorSubcoreMesh(core_axis_name="c", subcore_axis_name="s")


@functools.partial(
    pl.kernel,
    mesh=_MESH,
    out_type=[
        jax.ShapeDtypeStruct((16, 2 * _HALF), jnp.int32),
        jax.ShapeDtypeStruct((16, 2 * _HALF), jnp.int32),
    ],
    scratch_types=[
        pltpu.VMEM((_HALF,), jnp.int32),
        pltpu.VMEM((_HALF,), jnp.int32),
        pltpu.VMEM((_HALF,), jnp.int32),
        pltpu.SemaphoreType.DMA,
        pltpu.SemaphoreType.DMA,
    ],
    compiler_params=pltpu.CompilerParams(
        needs_layout_passes=False,
        skip_device_barrier=True,
        disable_bounds_checks=True,
        disable_semaphore_checks=True,
    ),
)
def _pack_sc(seg1, seg2, tok_out, sid_out, src_v, tok_v, sid_v, sem_in, sem_sid):
    half = lax.axis_index("c")
    row = lax.axis_index("s")
    col0 = half * _HALF

    @pl.when(half == 0)
    def _():
        pltpu.async_copy(seg1.at[row], src_v, sem_in)

    @pl.when(half == 1)
    def _():
        pltpu.async_copy(seg2.at[row], src_v, sem_in)

    lane = lax.iota(jnp.int32, _LANES)
    first_val = jnp.where(half == 0, jnp.int32(_START), jnp.int32(_SEP))
    is_second = (half == 1).astype(jnp.int32)

    sid_v[pl.ds(0, _LANES)] = jnp.where(lane == 0, jnp.int32(0), is_second)
    sid_fill = jnp.broadcast_to(is_second, (_LANES,))

    @plsc.parallel_loop(1, _CHUNKS, unroll=8)
    def _(j):
        sid_v[pl.ds(j * _LANES, _LANES)] = sid_fill

    sid_cp = pltpu.async_copy(sid_v, sid_out.at[row, pl.ds(col0, _HALF)], sem_sid)

    pltpu.make_async_copy(seg1.at[row], src_v, sem_in).wait()

    v0 = plsc.load_gather(src_v, [jnp.maximum(lane - 1, 0)])
    tok_v[pl.ds(0, _LANES)] = jnp.where(lane == 0, first_val, v0)

    @plsc.parallel_loop(1, _CHUNKS, unroll=8)
    def _(j):
        p = lane + j * _LANES
        v = plsc.load_gather(src_v, [p - 1])
        tok_v[pl.ds(j * _LANES, _LANES)] = v

    tail0 = _HALF - _LANES
    vt = tok_v[pl.ds(tail0, _LANES)]
    fix_end = (lane == _LANES - 1) & (half == 1)
    tok_v[pl.ds(tail0, _LANES)] = jnp.where(fix_end, jnp.int32(_END), vt)

    pltpu.sync_copy(tok_v, tok_out.at[row, pl.ds(col0, _HALF)])
    sid_cp.wait()


def kernel(seg1, seg2):
    tokens, segment_ids = _pack_sc(seg1, seg2)
    return tokens, segment_ids


# speedup vs baseline: 1.0136x; 1.0077x over previous
import functools

import jax
import jax.numpy as jnp
from jax import lax
from jax.experimental import pallas as pl
from jax.experimental.pallas import tpu as pltpu
from jax.experimental.pallas import tpu_sc as plsc

_START = 101
_END = 102
_SEP = 102
_HALF = 2048
_LANES = 16
_CHUNKS = _HALF // _LANES

_MESH = plsc.VectorSubcoreMesh(core_axis_name="c", subcore_axis_name="s")


@functools.partial(
    pl.kernel,
    mesh=_MESH,
    out_type=[
        jax.ShapeDtypeStruct((16, 2 * _HALF), jnp.int32),
        jax.ShapeDtypeStruct((16, 2 * _HALF), jnp.int32),
    ],
    scratch_types=[
        pltpu.VMEM((_HALF,), jnp.int32),
        pltpu.VMEM((_HALF,), jnp.int32),
        pltpu.VMEM((_HALF,), jnp.int32),
        pltpu.SemaphoreType.DMA,
        pltpu.SemaphoreType.DMA,
    ],
    compiler_params=pltpu.CompilerParams(
        needs_layout_passes=False, skip_device_barrier=True
    ),
)
def _pack_sc(seg1, seg2, tok_out, sid_out, src_v, tok_v, sid_v, sem_in, sem_sid):
    half = lax.axis_index("c")
    row = lax.axis_index("s")
    col0 = half * _HALF

    @pl.when(half == 0)
    def _():
        pltpu.async_copy(seg1.at[row], src_v, sem_in)

    @pl.when(half == 1)
    def _():
        pltpu.async_copy(seg2.at[row], src_v, sem_in)

    lane = lax.iota(jnp.int32, _LANES)
    first_val = jnp.where(half == 0, jnp.int32(_START), jnp.int32(_SEP))
    is_second = (half == 1).astype(jnp.int32)

    sid_v[pl.ds(0, _LANES)] = jnp.where(lane == 0, jnp.int32(0), is_second)
    sid_fill = jnp.broadcast_to(is_second, (_LANES,))

    @plsc.parallel_loop(1, _CHUNKS, unroll=8)
    def _(j):
        sid_v[pl.ds(j * _LANES, _LANES)] = sid_fill

    sid_cp = pltpu.async_copy(sid_v, sid_out.at[row, pl.ds(col0, _HALF)], sem_sid)

    pltpu.make_async_copy(seg1.at[row], src_v, sem_in).wait()

    v0 = plsc.load_gather(src_v, [jnp.maximum(lane - 1, 0)])
    tok_v[pl.ds(0, _LANES)] = jnp.where(lane == 0, first_val, v0)

    @plsc.parallel_loop(1, _CHUNKS, unroll=8)
    def _(j):
        v = src_v[pl.ds(j * _LANES - 1, _LANES)]
        tok_v[pl.ds(j * _LANES, _LANES)] = v

    tail0 = _HALF - _LANES
    vt = tok_v[pl.ds(tail0, _LANES)]
    fix_end = (lane == _LANES - 1) & (half == 1)
    tok_v[pl.ds(tail0, _LANES)] = jnp.where(fix_end, jnp.int32(_END), vt)

    pltpu.sync_copy(tok_v, tok_out.at[row, pl.ds(col0, _HALF)])
    sid_cp.wait()


def kernel(seg1, seg2):
    tokens, segment_ids = _pack_sc(seg1, seg2)
    return tokens, segment_ids
